# Initial kernel scaffold; baseline (speedup 1.0000x reference)
import jax
import jax.numpy as jnp
from jax.experimental import pallas as pl


def kernel(hidden_states, router_logits):
    raise NotImplementedError("write your pallas kernel here")


# speedup vs baseline: 13.1210x; 13.1210x over previous
import functools

import jax
import jax.numpy as jnp
from jax.experimental import pallas as pl
from jax.experimental.pallas import tpu as pltpu

NE = 16
T = 8192
D = 4096
BLK = 512


def _dispatch_combine_kernel(logits_ref, h_ref, out_ref, counts_ref):
    l = logits_ref[...]
    lane = jax.lax.broadcasted_iota(jnp.int32, l.shape, 1)

    l1 = jnp.max(l, axis=-1, keepdims=True)
    is1 = l == l1
    i1 = jnp.min(jnp.where(is1, lane, NE), axis=-1, keepdims=True)

    lm = jnp.where(lane == i1, -jnp.inf, l)
    l2 = jnp.max(lm, axis=-1, keepdims=True)
    is2 = lm == l2
    i2 = jnp.min(jnp.where(is2, lane, NE), axis=-1, keepdims=True)

    w1 = 1.0 / (1.0 + jnp.exp(l2 - l1))
    w2 = 1.0 / (1.0 + jnp.exp(l1 - l2))

    h = h_ref[...]
    out_ref[...] = h * w1 + h * w2

    hot = (lane == i1).astype(jnp.int32) + (lane == i2).astype(jnp.int32)
    part = jnp.sum(hot, axis=0, keepdims=True)

    @pl.when(pl.program_id(0) == 0)
    def _init():
        counts_ref[...] = part

    @pl.when(pl.program_id(0) != 0)
    def _acc():
        counts_ref[...] += part


@jax.jit
def kernel(hidden_states, router_logits):
    grid = (T // BLK,)
    combined, counts = pl.pallas_call(
        _dispatch_combine_kernel,
        grid=grid,
        in_specs=[
            pl.BlockSpec((BLK, NE), lambda i: (i, 0)),
            pl.BlockSpec((BLK, D), lambda i: (i, 0)),
        ],
        out_specs=[
            pl.BlockSpec((BLK, D), lambda i: (i, 0)),
            pl.BlockSpec((1, NE), lambda i: (0, 0)),
        ],
        out_shape=[
            jax.ShapeDtypeStruct((T, D), jnp.float32),
            jax.ShapeDtypeStruct((1, NE), jnp.int32),
        ],
        compiler_params=pltpu.CompilerParams(
            dimension_semantics=("arbitrary",),
        ),
    )(router_logits, hidden_states)
    return combined, counts.reshape(NE)
